# baseline (speedup 1.0000x reference)
import functools

import jax
import jax.numpy as jnp
from jax import lax
from jax.experimental import pallas as pl
from jax.experimental.pallas import tpu as pltpu
from jax.experimental.pallas import tpu_sc as plsc

_TD = 64
_SD = 32
_CHUNK = 128


@functools.lru_cache(maxsize=None)
def _make_gather(B, VT, VS):
    info = plsc.get_sparse_core_info()
    NC, NS = info.num_cores, info.num_subcores
    NW = NC * NS
    bpw = B // NW
    nchunks = bpw // _CHUNK

    mesh = plsc.VectorSubcoreMesh(core_axis_name="c", subcore_axis_name="s")

    @functools.partial(
        pl.kernel,
        mesh=mesh,
        out_type=(
            jax.ShapeDtypeStruct((B, _TD), jnp.float32),
            jax.ShapeDtypeStruct((B, _SD), jnp.float32),
        ),
        scratch_types=[
            pltpu.VMEM((bpw,), jnp.int32),
            pltpu.VMEM((bpw,), jnp.int32),
            pltpu.VMEM((_CHUNK, _TD), jnp.float32),
            pltpu.VMEM((_CHUNK, _TD), jnp.float32),
            pltpu.VMEM((_CHUNK, _SD), jnp.float32),
            pltpu.VMEM((_CHUNK, _SD), jnp.float32),
            pltpu.SemaphoreType.DMA,
            pltpu.SemaphoreType.DMA,
        ],
        compiler_params=pltpu.CompilerParams(use_tc_tiling_on_sc=False),
    )
    def gather(t_hbm, s_hbm, title_hbm, sub_hbm, out_t, out_s,
               ti_v, si_v, tr0, tr1, sr0, sr1, gsem, ssem):
        wid = lax.axis_index("s") * NC + lax.axis_index("c")
        base = wid * bpw
        hi = pltpu.async_copy(t_hbm.at[pl.ds(base, bpw)], ti_v, gsem)
        hs = pltpu.async_copy(s_hbm.at[pl.ds(base, bpw)], si_v, gsem)
        hi.wait()
        hs.wait()
        trows = (tr0, tr1)
        srows = (sr0, sr1)
        gh = []
        sh = []
        for j in range(nchunks):
            b = j % 2
            if j >= 2:
                sh[2 * (j - 2)].wait()
                sh[2 * (j - 2) + 1].wait()
            sl = pl.ds(j * _CHUNK, _CHUNK)
            gh.append(pltpu.async_copy(
                title_hbm.at[ti_v.at[sl]], trows[b], gsem))
            gh.append(pltpu.async_copy(
                sub_hbm.at[si_v.at[sl]], srows[b], gsem))
            gh[2 * j].wait()
            gh[2 * j + 1].wait()
            rows = pl.ds(base + j * _CHUNK, _CHUNK)
            sh.append(pltpu.async_copy(trows[b], out_t.at[rows], ssem))
            sh.append(pltpu.async_copy(srows[b], out_s.at[rows], ssem))
        for h in sh[-4:]:
            h.wait()

    return gather


def kernel(topic, subtopic, title_embed, subtopic_embed):
    B = topic.shape[0]
    t = topic.astype(jnp.int32)
    s = subtopic.astype(jnp.int32)
    out_t, out_s = _make_gather(
        B, title_embed.shape[0], subtopic_embed.shape[0])(
            t, s, title_embed, subtopic_embed)
    return jnp.concatenate([out_t, out_s], axis=1)


# speedup vs baseline: 1.0193x; 1.0193x over previous
import functools

import jax
import jax.numpy as jnp
from jax import lax
from jax.experimental import pallas as pl
from jax.experimental.pallas import tpu as pltpu
from jax.experimental.pallas import tpu_sc as plsc

_TD = 64
_SD = 32
_CHUNK = 128


@functools.lru_cache(maxsize=None)
def _make_gather(B, VT, VS):
    info = plsc.get_sparse_core_info()
    NC, NS = info.num_cores, info.num_subcores
    NW = NC * NS
    bpw = B // NW
    nchunks = bpw // _CHUNK

    mesh = plsc.VectorSubcoreMesh(core_axis_name="c", subcore_axis_name="s")

    @functools.partial(
        pl.kernel,
        mesh=mesh,
        out_type=jax.ShapeDtypeStruct((B, _TD + _SD), jnp.float32),
        scratch_types=[
            pltpu.VMEM((bpw,), jnp.int32),
            pltpu.VMEM((bpw,), jnp.int32),
            pltpu.VMEM((_CHUNK, _TD), jnp.float32),
            pltpu.VMEM((_CHUNK, _TD), jnp.float32),
            pltpu.VMEM((_CHUNK, _SD), jnp.float32),
            pltpu.VMEM((_CHUNK, _SD), jnp.float32),
            pltpu.SemaphoreType.DMA,
            pltpu.SemaphoreType.DMA,
        ],
        compiler_params=pltpu.CompilerParams(use_tc_tiling_on_sc=False),
    )
    def gather(t_hbm, s_hbm, title_hbm, sub_hbm, out,
               ti_v, si_v, tr0, tr1, sr0, sr1, gsem, ssem):
        wid = lax.axis_index("s") * NC + lax.axis_index("c")
        base = wid * bpw
        hi = pltpu.async_copy(t_hbm.at[pl.ds(base, bpw)], ti_v, gsem)
        hs = pltpu.async_copy(s_hbm.at[pl.ds(base, bpw)], si_v, gsem)
        hi.wait()
        hs.wait()
        trows = (tr0, tr1)
        srows = (sr0, sr1)
        gh = []
        sh = []
        for j in range(nchunks):
            b = j % 2
            sl = pl.ds(j * _CHUNK, _CHUNK)
            if j >= 2:
                sh[2 * (j - 2)].wait()
                sh[2 * (j - 2) + 1].wait()
            gh.append(pltpu.async_copy(
                title_hbm.at[ti_v.at[sl]], trows[b], gsem))
            gh.append(pltpu.async_copy(
                sub_hbm.at[si_v.at[sl]], srows[b], gsem))
            if j >= 1:
                gh[2 * (j - 1)].wait()
                gh[2 * (j - 1) + 1].wait()
                rows = pl.ds(base + (j - 1) * _CHUNK, _CHUNK)
                sh.append(pltpu.async_copy(
                    trows[1 - b], out.at[rows, pl.ds(0, _TD)], ssem))
                sh.append(pltpu.async_copy(
                    srows[1 - b], out.at[rows, pl.ds(_TD, _SD)], ssem))
        j = nchunks - 1
        gh[2 * j].wait()
        gh[2 * j + 1].wait()
        rows = pl.ds(base + j * _CHUNK, _CHUNK)
        sh.append(pltpu.async_copy(
            trows[j % 2], out.at[rows, pl.ds(0, _TD)], ssem))
        sh.append(pltpu.async_copy(
            srows[j % 2], out.at[rows, pl.ds(_TD, _SD)], ssem))
        for h in sh[-4:]:
            h.wait()

    return gather


def kernel(topic, subtopic, title_embed, subtopic_embed):
    B = topic.shape[0]
    t = topic.astype(jnp.int32)
    s = subtopic.astype(jnp.int32)
    return _make_gather(
        B, title_embed.shape[0], subtopic_embed.shape[0])(
            t, s, title_embed, subtopic_embed)
